# Initial kernel scaffold; baseline (speedup 1.0000x reference)
import jax
import jax.numpy as jnp
from jax.experimental import pallas as pl


def kernel(s, token_to_atom_idx, W, b):
    raise NotImplementedError("write your pallas kernel here")


# speedup vs baseline: 44.0418x; 44.0418x over previous
import jax
import jax.numpy as jnp
from jax.experimental import pallas as pl


def _fused_kernel(s_ref, t_ref, w_ref, b_ref, out_ref):
    sw = jnp.dot(s_ref[:], w_ref[:].T, preferred_element_type=jnp.float32)
    out_ref[:] = (
        jnp.dot(t_ref[:], sw, preferred_element_type=jnp.float32) + b_ref[:]
    )


def kernel(s, token_to_atom_idx, W, b):
    B, n_res, c_s = s.shape
    _, n_atom, _ = token_to_atom_idx.shape
    c_out = W.shape[0]
    out = pl.pallas_call(
        _fused_kernel,
        out_shape=jax.ShapeDtypeStruct((n_atom, c_out), jnp.float32),
    )(
        s.reshape(n_res, c_s),
        token_to_atom_idx.reshape(n_atom, n_res),
        W,
        b.reshape(1, c_out),
    )
    return out.reshape(B, n_atom, c_out)
